# baseline (speedup 1.0000x reference)
import jax
import jax.numpy as jnp
from jax import lax
from jax.experimental import pallas as pl
from jax.experimental.pallas import tpu as pltpu
from jax.experimental.pallas import tpu_sc as plsc

N = 10000
E = 160000
EPAD = 163840
DF = 256
DH = 128
DE = 16
DKQ = 64
RH = 16
NC = 2
NS = 16
NBLK_N = 10
RB = N // NBLK_N
EBLK = 1280
NEB = EPAD // EBLK
NEB_REAL = E // EBLK
NPS_A = 624
NPS_L = N - (NS - 1) * NPS_A
NS8 = N // 8
DKQ2 = 2 * DKQ

_f32 = jnp.float32


def _node_proj_body(x, wk, wv, wq1, bq1, wq2, bq2, kq, v):
    xx = x[...]
    kq[:, :DKQ] = jnp.dot(xx, wk[...], preferred_element_type=_f32)
    v[...] = jnp.dot(xx, wv[...], preferred_element_type=_f32)
    h = jnp.maximum(jnp.dot(xx, wq1[...], preferred_element_type=_f32) + bq1[...], 0.0)
    kq[:, DKQ:] = jnp.dot(h, wq2[...], preferred_element_type=_f32) + bq2[...]


def _node_proj(x, wk, wv, wq1, bq1, wq2, bq2, interpret=False):
    full = lambda r, c: pl.BlockSpec((r, c), lambda i: (0, 0))
    return pl.pallas_call(
        _node_proj_body,
        grid=(NBLK_N,),
        in_specs=[pl.BlockSpec((RB, DF), lambda i: (i, 0)),
                  full(DF, DKQ), full(DF, DF), full(DF, RH), full(1, RH),
                  full(RH, DKQ), full(1, DKQ)],
        out_specs=[pl.BlockSpec((RB, DKQ2), lambda i: (i, 0)),
                   pl.BlockSpec((RB, DF), lambda i: (i, 0))],
        out_shape=[jax.ShapeDtypeStruct((N, DKQ2), _f32),
                   jax.ShapeDtypeStruct((N, DF), _f32)],
        interpret=interpret,
    )(x, wk, wv, wq1, bq1, wq2, bq2)


CHK = 128
CHV = 64
NCHK = EPAD // CHK // NS
NCHV = EPAD // CHV // NS


def _gather_body(kq_hbm, v_hbm, src_hbm, dst_hbm,
                 ks_hbm, qd_hbm, vs_hbm,
                 srcv, dstv, kqsb, kqdb, srcv2, vsb, gsem, wsem):
    cid = lax.axis_index("c")
    sid = lax.axis_index("s")

    def kq_fire(c, b):
        e0 = pl.multiple_of((sid * NCHK + c) * CHK, CHK)
        pltpu.sync_copy(src_hbm.at[pl.ds(e0, CHK)], srcv.at[b])
        pltpu.async_copy(kq_hbm.at[srcv.at[b]], kqsb.at[b], gsem.at[b])

        @pl.when(c < NCHK // 2)
        def _():
            pltpu.sync_copy(dst_hbm.at[pl.ds(e0, CHK)], dstv.at[b])
            pltpu.async_copy(kq_hbm.at[dstv.at[b]], kqdb.at[b], gsem.at[b])

    def kq_wait_gather(c, b):
        pltpu.make_async_copy(kq_hbm.at[srcv.at[b]], kqsb.at[b], gsem.at[b]).wait()

        @pl.when(c < NCHK // 2)
        def _():
            pltpu.make_async_copy(kq_hbm.at[dstv.at[b]], kqdb.at[b], gsem.at[b]).wait()

    def kq_start_wb(c, b):
        e0 = pl.multiple_of((sid * NCHK + c) * CHK, CHK)
        pltpu.async_copy(kqsb.at[b], ks_hbm.at[pl.ds(e0, CHK)], wsem.at[b])

        @pl.when(c < NCHK // 2)
        def _():
            pltpu.async_copy(kqdb.at[b], qd_hbm.at[pl.ds(e0, CHK)], wsem.at[b])

    def kq_wait_wb(c, b):
        pltpu.make_async_copy(kqsb.at[b], ks_hbm.at[pl.ds(0, CHK)], wsem.at[b]).wait()

        @pl.when(c < NCHK // 2)
        def _():
            pltpu.make_async_copy(kqdb.at[b], qd_hbm.at[pl.ds(0, CHK)], wsem.at[b]).wait()

    def v_fire(c, b):
        e0 = pl.multiple_of((sid * NCHV + c) * CHV, CHV)
        pltpu.sync_copy(src_hbm.at[pl.ds(e0, CHV)], srcv2.at[b])
        pltpu.async_copy(v_hbm.at[srcv2.at[b]], vsb.at[b], gsem.at[b])

        @pl.when((c % 2 == 0) & (c < NCHV // 2))
        def _():
            d0 = pl.multiple_of((sid * NCHK + NCHK // 2 + c // 2) * CHK, CHK)
            pltpu.sync_copy(dst_hbm.at[pl.ds(d0, CHK)], dstv.at[b])
            pltpu.async_copy(kq_hbm.at[dstv.at[b]], kqdb.at[b], gsem.at[b])

    def v_wait_gather(c, b):
        pltpu.make_async_copy(v_hbm.at[srcv2.at[b]], vsb.at[b], gsem.at[b]).wait()

        @pl.when((c % 2 == 0) & (c < NCHV // 2))
        def _():
            pltpu.make_async_copy(kq_hbm.at[dstv.at[b]], kqdb.at[b], gsem.at[b]).wait()

    def v_start_wb(c, b):
        e0 = pl.multiple_of((sid * NCHV + c) * CHV, CHV)
        pltpu.async_copy(vsb.at[b], vs_hbm.at[pl.ds(e0, CHV)], wsem.at[b])

        @pl.when((c % 2 == 0) & (c < NCHV // 2))
        def _():
            d0 = pl.multiple_of((sid * NCHK + NCHK // 2 + c // 2) * CHK, CHK)
            pltpu.async_copy(kqdb.at[b], qd_hbm.at[pl.ds(d0, CHK)], wsem.at[b])

    def v_wait_wb(c, b):
        pltpu.make_async_copy(vsb.at[b], vs_hbm.at[pl.ds(0, CHV)], wsem.at[b]).wait()

        @pl.when((c % 2 == 0) & (c < NCHV // 2))
        def _():
            pltpu.make_async_copy(kqdb.at[b], qd_hbm.at[pl.ds(0, CHK)], wsem.at[b]).wait()

    def pipeline(nch, fire, wait_gather, start_wb, wait_wb):
        fire(0, 0)
        fire(1, 1)

        @pl.loop(0, nch, step=2)
        def _(k):
            for b in (0, 1):
                c = k + b
                wait_gather(c, b)
                start_wb(c, b)

                @pl.when(c + 2 < nch)
                def _():
                    wait_wb(c, b)
                    fire(c + 2, b)

        wait_wb(nch - 2, 0)
        wait_wb(nch - 1, 1)

    @pl.when(cid == 0)
    def _():
        pipeline(NCHK, kq_fire, kq_wait_gather, kq_start_wb, kq_wait_wb)

    @pl.when(cid == 1)
    def _():
        pipeline(NCHV, v_fire, v_wait_gather, v_start_wb, v_wait_wb)


def _sc_gather(kq, v, src, dst):
    mesh = plsc.VectorSubcoreMesh(core_axis_name="c", subcore_axis_name="s",
                                  num_cores=NC, num_subcores=NS)
    f = pl.kernel(
        _gather_body,
        out_type=[jax.ShapeDtypeStruct((EPAD, DKQ2), _f32),
                  jax.ShapeDtypeStruct((EPAD, DKQ2), _f32),
                  jax.ShapeDtypeStruct((EPAD, DF), _f32)],
        mesh=mesh,
        scratch_types=[pltpu.VMEM((2, CHK), jnp.int32),
                       pltpu.VMEM((2, CHK), jnp.int32),
                       pltpu.VMEM((2, CHK, DKQ2), _f32),
                       pltpu.VMEM((2, CHK, DKQ2), _f32),
                       pltpu.VMEM((2, CHV), jnp.int32),
                       pltpu.VMEM((2, CHV, DF), _f32),
                       pltpu.SemaphoreType.DMA((2,)),
                       pltpu.SemaphoreType.DMA((2,))],
    )
    return f(kq, v, src, dst)


def _msg_body(ef, ks, qd, vs, dst3, wrk1, brk1, wrk2, brk2,
              wrv1, brv1, wrv2, brv2, mlo, mhi, eb):
    i = pl.program_id(0)
    efv = ef[...]
    h = jnp.tanh(jnp.dot(efv, wrk1[...], preferred_element_type=_f32) + brk1[...])
    rk = jnp.dot(h, wrk2[...], preferred_element_type=_f32) + brk2[...]
    l = jnp.sum(ks[:, :DKQ] * rk * qd[:, DKQ:], axis=1) * 0.125
    e = jnp.exp(jnp.minimum(l, 60.0))
    e = jnp.where(i < NEB_REAL, e, jnp.zeros_like(e))
    h2 = jnp.tanh(jnp.dot(efv, wrv1[...], preferred_element_type=_f32) + brv1[...])
    rv = jnp.dot(h2, wrv2[...], preferred_element_type=_f32) + brv2[...]
    m = (e[:, None] * rv) * vs[...]
    mlo[...] = m[:, :DH]
    mhi[...] = m[:, DH:]
    d = dst3[...].reshape(EBLK)
    slot = jax.lax.broadcasted_iota(jnp.int32, (EBLK, DKQ2), 1) // DE
    eb[...] = jnp.where(slot == (d % 8)[:, None], e[:, None], 0.0)


def _edge_msg(ef, ks, qd, vs, dst3, wrk1, brk1, wrk2, brk2,
              wrv1, brv1, wrv2, brv2, interpret=False):
    full = lambda r, c: pl.BlockSpec((r, c), lambda i: (0, 0))
    return pl.pallas_call(
        _msg_body,
        grid=(NEB,),
        in_specs=[pl.BlockSpec((EBLK, DE), lambda i: (i, 0)),
                  pl.BlockSpec((EBLK, DKQ2), lambda i: (i, 0)),
                  pl.BlockSpec((EBLK, DKQ2), lambda i: (i, 0)),
                  pl.BlockSpec((EBLK, DF), lambda i: (i, 0)),
                  pl.BlockSpec((1, 1, EBLK), lambda i: (i, 0, 0)),
                  full(DE, RH), full(1, RH), full(RH, DKQ), full(1, DKQ),
                  full(DE, RH), full(1, RH), full(RH, DF), full(1, DF)],
        out_specs=[pl.BlockSpec((EBLK, DH), lambda i: (i, 0)),
                   pl.BlockSpec((EBLK, DH), lambda i: (i, 0)),
                   pl.BlockSpec((EBLK, DKQ2), lambda i: (i, 0))],
        out_shape=[jax.ShapeDtypeStruct((EPAD, DH), _f32),
                   jax.ShapeDtypeStruct((EPAD, DH), _f32),
                   jax.ShapeDtypeStruct((EPAD, DKQ2), _f32)],
        interpret=interpret,
    )(ef, ks, qd, vs, dst3, wrk1, brk1, wrk2, brk2, wrv1, brv1, wrv2, brv2)


CHS = 64
NCHS = EPAD // CHS // NS
NCHE = NCHS // 2


def _scatter_body(mlo_hbm, mhi_hbm, eb_hbm, dst_hbm, zrow_hbm, zs_hbm,
                  outlo_hbm, outhi_hbm, s0_hbm, s1_hbm,
                  out_sh, s_sh, dstv, dstv8, mb, ev, lsem, ssem):
    cid = lax.axis_index("c")
    sid = lax.axis_index("s")
    rows0 = pl.multiple_of(sid * NPS_A, 8)

    @pl.when(sid < NS - 1)
    def _():
        pltpu.sync_copy(zrow_hbm.at[pl.ds(0, NPS_A)],
                        out_sh.at[pl.ds(rows0, NPS_A)])

    @pl.when(sid == NS - 1)
    def _():
        pltpu.sync_copy(zrow_hbm, out_sh.at[pl.ds((NS - 1) * NPS_A, NPS_L)])

    @pl.when(sid == 0)
    def _():
        pltpu.sync_copy(zs_hbm, s_sh)

    plsc.subcore_barrier()

    def body(m_hbm):
        def mload(c, b):
            e0 = pl.multiple_of((sid * NCHS + c) * CHS, CHS)
            pltpu.sync_copy(dst_hbm.at[pl.ds(e0, CHS)], dstv.at[b])
            pltpu.async_copy(m_hbm.at[pl.ds(e0, CHS)], mb.at[b], lsem.at[b])

        def mwait_load(b):
            pltpu.make_async_copy(m_hbm.at[pl.ds(0, CHS)], mb.at[b],
                                  lsem.at[b]).wait()

        def mscat(b):
            pltpu.async_copy(mb.at[b], out_sh.at[dstv.at[b]], ssem.at[b],
                             add=True)

        def mwait_scat(b):
            pltpu.make_async_copy(mb.at[b], out_sh.at[pl.ds(0, CHS)],
                                  ssem.at[b]).wait()

        mload(0, 0)
        mload(1, 1)

        @pl.loop(0, NCHS, step=2)
        def _(k):
            for b in (0, 1):
                c = k + b
                mwait_load(b)
                mscat(b)

                @pl.when(c + 2 < NCHS)
                def _():
                    mwait_scat(b)
                    mload(c + 2, b)

        mwait_scat(0)
        mwait_scat(1)

        def eload(c, b):
            e0 = pl.multiple_of((cid * NS * NCHE + sid * NCHE + c) * CHS, CHS)
            pltpu.sync_copy(dst_hbm.at[pl.ds(e0, CHS)], dstv.at[b])
            for j in range(CHS // 16):
                dstv8[b, pl.ds(j * 16, 16)] = lax.shift_right_logical(
                    dstv[b, pl.ds(j * 16, 16)], 3)
            pltpu.async_copy(eb_hbm.at[pl.ds(e0, CHS)], ev.at[b], lsem.at[b])

        def ewait_load(b):
            pltpu.make_async_copy(eb_hbm.at[pl.ds(0, CHS)], ev.at[b],
                                  lsem.at[b]).wait()

        def escat(b):
            pltpu.async_copy(ev.at[b], s_sh.at[dstv8.at[b]], ssem.at[b],
                             add=True)

        def ewait_scat(b):
            pltpu.make_async_copy(ev.at[b], s_sh.at[pl.ds(0, CHS)],
                                  ssem.at[b]).wait()

        eload(0, 0)
        eload(1, 1)

        @pl.loop(0, NCHE, step=2)
        def _(k):
            for b in (0, 1):
                c = k + b
                ewait_load(b)
                escat(b)

                @pl.when(c + 2 < NCHE)
                def _():
                    ewait_scat(b)
                    eload(c + 2, b)

        ewait_scat(0)
        ewait_scat(1)

    @pl.when(cid == 0)
    def _():
        body(mlo_hbm)

    @pl.when(cid == 1)
    def _():
        body(mhi_hbm)

    plsc.subcore_barrier()

    def drain(dst_hbm_arr):
        @pl.when(sid < NS - 1)
        def _():
            pltpu.sync_copy(out_sh.at[pl.ds(rows0, NPS_A)],
                            dst_hbm_arr.at[pl.ds(rows0, NPS_A)])

        @pl.when(sid == NS - 1)
        def _():
            pltpu.sync_copy(out_sh.at[pl.ds((NS - 1) * NPS_A, NPS_L)],
                            dst_hbm_arr.at[pl.ds((NS - 1) * NPS_A, NPS_L)])

    @pl.when(cid == 0)
    def _():
        drain(outlo_hbm)

    @pl.when((cid == 0) & (sid == 0))
    def _():
        pltpu.sync_copy(s_sh, s0_hbm)

    @pl.when(cid == 1)
    def _():
        drain(outhi_hbm)

    @pl.when((cid == 1) & (sid == 0))
    def _():
        pltpu.sync_copy(s_sh, s1_hbm)


def _sc_scatter(mlo, mhi, eb, dst, z_row, z_s):
    mesh = plsc.VectorSubcoreMesh(core_axis_name="c", subcore_axis_name="s",
                                  num_cores=NC, num_subcores=NS)
    f = pl.kernel(
        _scatter_body,
        out_type=[jax.ShapeDtypeStruct((N, DH), _f32),
                  jax.ShapeDtypeStruct((N, DH), _f32),
                  jax.ShapeDtypeStruct((NS8, DKQ2), _f32),
                  jax.ShapeDtypeStruct((NS8, DKQ2), _f32)],
        mesh=mesh,
        scratch_types=[pltpu.VMEM_SHARED((N, DH), _f32),
                       pltpu.VMEM_SHARED((NS8, DKQ2), _f32),
                       pltpu.VMEM((2, CHS), jnp.int32),
                       pltpu.VMEM((2, CHS), jnp.int32),
                       pltpu.VMEM((2, CHS, DH), _f32),
                       pltpu.VMEM((2, CHS, DKQ2), _f32),
                       pltpu.SemaphoreType.DMA((2,)),
                       pltpu.SemaphoreType.DMA((2,))],
    )
    return f(mlo, mhi, eb, dst, z_row, z_s)


def _norm_body(lo, hi, s0, s1, x):
    inv = 1.0 / (s0[...][:, :1] + s1[...][:, :1] + 1e-9)
    x[:, :DH] = lo[...] * inv
    x[:, DH:] = hi[...] * inv


def _normalize(lo, hi, s0, s1, interpret=False):
    return pl.pallas_call(
        _norm_body,
        grid=(NBLK_N,),
        in_specs=[pl.BlockSpec((RB, DH), lambda i: (i, 0)),
                  pl.BlockSpec((RB, DH), lambda i: (i, 0)),
                  pl.BlockSpec((RB, DE), lambda i: (i, 0)),
                  pl.BlockSpec((RB, DE), lambda i: (i, 0))],
        out_specs=pl.BlockSpec((RB, DF), lambda i: (i, 0)),
        out_shape=jax.ShapeDtypeStruct((N, DF), _f32),
        interpret=interpret,
    )(lo, hi, s0, s1)


def _layer(x, ef_pad, src, dst, dst3, z_row, z_s, p):
    (Wk, Wv, Wrk1, brk1, Wrk2, brk2, Wrv1, brv1, Wrv2, brv2,
     Wq1, bq1, Wq2, bq2) = p
    KQ, V = _node_proj(x, Wk, Wv, Wq1, bq1.reshape(1, -1), Wq2,
                       bq2.reshape(1, -1))
    KQS, KQD, VS = _sc_gather(KQ, V, src, dst)
    Mlo, Mhi, eb = _edge_msg(ef_pad, KQS, KQD, VS, dst3,
                             Wrk1, brk1.reshape(1, -1), Wrk2, brk2.reshape(1, -1),
                             Wrv1, brv1.reshape(1, -1), Wrv2, brv2.reshape(1, -1))
    OL, OH, S0, S1 = _sc_scatter(Mlo, Mhi, eb, dst, z_row, z_s)
    return _normalize(OL, OH, S0.reshape(N, DE), S1.reshape(N, DE))


def kernel(node_features, edge_index, edge_features,
           Wk_0, Wv_0, Wrk1_0, brk1_0, Wrk2_0, brk2_0, Wrv1_0, brv1_0,
           Wrv2_0, brv2_0, Wq1_0, bq1_0, Wq2_0, bq2_0,
           Wk_1, Wv_1, Wrk1_1, brk1_1, Wrk2_1, brk2_1, Wrv1_1, brv1_1,
           Wrv2_1, brv2_1, Wq1_1, bq1_1, Wq2_1, bq2_1):
    ei_pad = jnp.pad(edge_index, ((0, 0), (0, EPAD - E)))
    src, dst = ei_pad[0], ei_pad[1]
    dst3 = dst.reshape(NEB, 1, EBLK)
    ef_pad = jnp.pad(edge_features, ((0, EPAD - E), (0, 0)))
    z_row = jnp.zeros((NPS_L, DH), _f32)
    z_s = jnp.zeros((NS8, DKQ2), _f32)
    layer_params = [
        [Wk_0, Wv_0, Wrk1_0, brk1_0, Wrk2_0, brk2_0, Wrv1_0, brv1_0,
         Wrv2_0, brv2_0, Wq1_0, bq1_0, Wq2_0, bq2_0],
        [Wk_1, Wv_1, Wrk1_1, brk1_1, Wrk2_1, brk2_1, Wrv1_1, brv1_1,
         Wrv2_1, brv2_1, Wq1_1, bq1_1, Wq2_1, bq2_1],
    ]
    x = node_features
    for p in layer_params:
        x = _layer(x, ef_pad, src, dst, dst3, z_row, z_s, p)
    return x


# speedup vs baseline: 1.2088x; 1.2088x over previous
import jax
import jax.numpy as jnp
from jax import lax
from jax.experimental import pallas as pl
from jax.experimental.pallas import tpu as pltpu
from jax.experimental.pallas import tpu_sc as plsc

N = 10000
E = 160000
EPAD = 163840
DF = 256
DH = 128
DE = 16
DKQ = 64
RH = 16
NC = 2
NS = 16
NBLK_N = 10
RB = N // NBLK_N
EBLK = 1280
NEB = EPAD // EBLK
NEB_REAL = E // EBLK
NPS_A = 624
NPS_L = N - (NS - 1) * NPS_A
NS8 = N // 8
DKQ2 = 2 * DKQ

_f32 = jnp.float32


def _node_proj_body(x, wk, wv, wq1, bq1, wq2, bq2, kq, v):
    xx = x[...]
    kq[:, :DKQ] = jnp.dot(xx, wk[...], preferred_element_type=_f32)
    v[...] = jnp.dot(xx, wv[...], preferred_element_type=_f32)
    h = jnp.maximum(jnp.dot(xx, wq1[...], preferred_element_type=_f32) + bq1[...], 0.0)
    kq[:, DKQ:] = jnp.dot(h, wq2[...], preferred_element_type=_f32) + bq2[...]


def _node_proj(x, wk, wv, wq1, bq1, wq2, bq2, interpret=False):
    full = lambda r, c: pl.BlockSpec((r, c), lambda i: (0, 0))
    return pl.pallas_call(
        _node_proj_body,
        grid=(NBLK_N,),
        in_specs=[pl.BlockSpec((RB, DF), lambda i: (i, 0)),
                  full(DF, DKQ), full(DF, DF), full(DF, RH), full(1, RH),
                  full(RH, DKQ), full(1, DKQ)],
        out_specs=[pl.BlockSpec((RB, DKQ2), lambda i: (i, 0)),
                   pl.BlockSpec((RB, DF), lambda i: (i, 0))],
        out_shape=[jax.ShapeDtypeStruct((N, DKQ2), _f32),
                   jax.ShapeDtypeStruct((N, DF), _f32)],
        interpret=interpret,
    )(x, wk, wv, wq1, bq1, wq2, bq2)


CHK = 128
CHV = 64
NCHK = EPAD // CHK // NS
NCHV = EPAD // CHV // NS


def _gather_body(kq_hbm, v_hbm, src_hbm, dst_hbm,
                 ks_hbm, qd_hbm, vs_hbm,
                 srcv, dstv, kqsb, kqdb, srcv2, vsb, gsem, wsem):
    cid = lax.axis_index("c")
    sid = lax.axis_index("s")

    def kq_fire(c, b):
        e0 = pl.multiple_of((sid * NCHK + c) * CHK, CHK)
        pltpu.sync_copy(src_hbm.at[pl.ds(e0, CHK)], srcv.at[b])
        pltpu.sync_copy(dst_hbm.at[pl.ds(e0, CHK)], dstv.at[b])
        pltpu.async_copy(kq_hbm.at[srcv.at[b]], kqsb.at[b], gsem.at[b])
        pltpu.async_copy(kq_hbm.at[dstv.at[b]], kqdb.at[b], gsem.at[b])

    def kq_wait_gather(b):
        pltpu.make_async_copy(kq_hbm.at[srcv.at[b]], kqsb.at[b], gsem.at[b]).wait()
        pltpu.make_async_copy(kq_hbm.at[dstv.at[b]], kqdb.at[b], gsem.at[b]).wait()

    def kq_start_wb(c, b):
        e0 = pl.multiple_of((sid * NCHK + c) * CHK, CHK)
        pltpu.async_copy(kqsb.at[b], ks_hbm.at[pl.ds(e0, CHK)], wsem.at[b])
        pltpu.async_copy(kqdb.at[b], qd_hbm.at[pl.ds(e0, CHK)], wsem.at[b])

    def kq_wait_wb(b):
        pltpu.make_async_copy(kqsb.at[b], ks_hbm.at[pl.ds(0, CHK)], wsem.at[b]).wait()
        pltpu.make_async_copy(kqdb.at[b], qd_hbm.at[pl.ds(0, CHK)], wsem.at[b]).wait()

    def v_fire(c, b):
        e0 = pl.multiple_of((sid * NCHV + c) * CHV, CHV)
        pltpu.sync_copy(src_hbm.at[pl.ds(e0, CHV)], srcv2.at[b])
        pltpu.async_copy(v_hbm.at[srcv2.at[b]], vsb.at[b], gsem.at[b])

    def v_wait_gather(b):
        pltpu.make_async_copy(v_hbm.at[srcv2.at[b]], vsb.at[b], gsem.at[b]).wait()

    def v_start_wb(c, b):
        e0 = pl.multiple_of((sid * NCHV + c) * CHV, CHV)
        pltpu.async_copy(vsb.at[b], vs_hbm.at[pl.ds(e0, CHV)], wsem.at[b])

    def v_wait_wb(b):
        pltpu.make_async_copy(vsb.at[b], vs_hbm.at[pl.ds(0, CHV)], wsem.at[b]).wait()

    def pipeline(nch, fire, wait_gather, start_wb, wait_wb):
        fire(0, 0)
        fire(1, 1)

        @pl.loop(0, nch, step=2)
        def _(k):
            for b in (0, 1):
                c = k + b
                wait_gather(b)
                start_wb(c, b)

                @pl.when(c + 2 < nch)
                def _():
                    wait_wb(b)
                    fire(c + 2, b)

        wait_wb(0)
        wait_wb(1)

    @pl.when(cid == 0)
    def _():
        pipeline(NCHK, kq_fire, kq_wait_gather, kq_start_wb, kq_wait_wb)

    @pl.when(cid == 1)
    def _():
        pipeline(NCHV, v_fire, v_wait_gather, v_start_wb, v_wait_wb)


def _sc_gather(kq, v, src, dst):
    mesh = plsc.VectorSubcoreMesh(core_axis_name="c", subcore_axis_name="s",
                                  num_cores=NC, num_subcores=NS)
    f = pl.kernel(
        _gather_body,
        out_type=[jax.ShapeDtypeStruct((EPAD, DKQ2), _f32),
                  jax.ShapeDtypeStruct((EPAD, DKQ2), _f32),
                  jax.ShapeDtypeStruct((EPAD, DF), _f32)],
        mesh=mesh,
        scratch_types=[pltpu.VMEM((2, CHK), jnp.int32),
                       pltpu.VMEM((2, CHK), jnp.int32),
                       pltpu.VMEM((2, CHK, DKQ2), _f32),
                       pltpu.VMEM((2, CHK, DKQ2), _f32),
                       pltpu.VMEM((2, CHV), jnp.int32),
                       pltpu.VMEM((2, CHV, DF), _f32),
                       pltpu.SemaphoreType.DMA((2,)),
                       pltpu.SemaphoreType.DMA((2,))],
    )
    return f(kq, v, src, dst)


def _msg_body(ef, ks, qd, vs, dst3, wrk1, brk1, wrk2, brk2,
              wrv1, brv1, wrv2, brv2, mlo, mhi, eb):
    i = pl.program_id(0)
    efv = ef[...]
    h = jnp.tanh(jnp.dot(efv, wrk1[...], preferred_element_type=_f32) + brk1[...])
    rk = jnp.dot(h, wrk2[...], preferred_element_type=_f32) + brk2[...]
    l = jnp.sum(ks[:, :DKQ] * rk * qd[:, DKQ:], axis=1) * 0.125
    e = jnp.exp(jnp.minimum(l, 60.0))
    e = jnp.where(i < NEB_REAL, e, jnp.zeros_like(e))
    h2 = jnp.tanh(jnp.dot(efv, wrv1[...], preferred_element_type=_f32) + brv1[...])
    rv = jnp.dot(h2, wrv2[...], preferred_element_type=_f32) + brv2[...]
    m = (e[:, None] * rv) * vs[...]
    mlo[...] = m[:, :DH]
    mhi[...] = m[:, DH:]
    d = dst3[...].reshape(EBLK)
    slot = jax.lax.broadcasted_iota(jnp.int32, (EBLK, DKQ2), 1) // DE
    eb[...] = jnp.where(slot == (d % 8)[:, None], e[:, None], 0.0)


def _edge_msg(ef, ks, qd, vs, dst3, wrk1, brk1, wrk2, brk2,
              wrv1, brv1, wrv2, brv2, interpret=False):
    full = lambda r, c: pl.BlockSpec((r, c), lambda i: (0, 0))
    return pl.pallas_call(
        _msg_body,
        grid=(NEB,),
        in_specs=[pl.BlockSpec((EBLK, DE), lambda i: (i, 0)),
                  pl.BlockSpec((EBLK, DKQ2), lambda i: (i, 0)),
                  pl.BlockSpec((EBLK, DKQ2), lambda i: (i, 0)),
                  pl.BlockSpec((EBLK, DF), lambda i: (i, 0)),
                  pl.BlockSpec((1, 1, EBLK), lambda i: (i, 0, 0)),
                  full(DE, RH), full(1, RH), full(RH, DKQ), full(1, DKQ),
                  full(DE, RH), full(1, RH), full(RH, DF), full(1, DF)],
        out_specs=[pl.BlockSpec((EBLK, DH), lambda i: (i, 0)),
                   pl.BlockSpec((EBLK, DH), lambda i: (i, 0)),
                   pl.BlockSpec((EBLK, DKQ2), lambda i: (i, 0))],
        out_shape=[jax.ShapeDtypeStruct((EPAD, DH), _f32),
                   jax.ShapeDtypeStruct((EPAD, DH), _f32),
                   jax.ShapeDtypeStruct((EPAD, DKQ2), _f32)],
        interpret=interpret,
    )(ef, ks, qd, vs, dst3, wrk1, brk1, wrk2, brk2, wrv1, brv1, wrv2, brv2)


CHS = 64
NCHS = EPAD // CHS // NS
NCHE = NCHS // 2


def _scatter_body(mlo_hbm, mhi_hbm, eb_hbm, dst_hbm, zrow_hbm, zs_hbm,
                  outlo_hbm, outhi_hbm, s0_hbm, s1_hbm,
                  out_sh, s_sh, dstv, dstv8, mb, ev, lsem, ssem):
    cid = lax.axis_index("c")
    sid = lax.axis_index("s")
    rows0 = pl.multiple_of(sid * NPS_A, 8)

    @pl.when(sid < NS - 1)
    def _():
        pltpu.sync_copy(zrow_hbm.at[pl.ds(0, NPS_A)],
                        out_sh.at[pl.ds(rows0, NPS_A)])

    @pl.when(sid == NS - 1)
    def _():
        pltpu.sync_copy(zrow_hbm, out_sh.at[pl.ds((NS - 1) * NPS_A, NPS_L)])

    @pl.when(sid == 0)
    def _():
        pltpu.sync_copy(zs_hbm, s_sh)

    plsc.subcore_barrier()

    def body(m_hbm):
        def mload(c, b):
            e0 = pl.multiple_of((sid * NCHS + c) * CHS, CHS)
            pltpu.sync_copy(dst_hbm.at[pl.ds(e0, CHS)], dstv.at[b])
            pltpu.async_copy(m_hbm.at[pl.ds(e0, CHS)], mb.at[b], lsem.at[b])

        def mwait_load(b):
            pltpu.make_async_copy(m_hbm.at[pl.ds(0, CHS)], mb.at[b],
                                  lsem.at[b]).wait()

        def mscat(b):
            pltpu.async_copy(mb.at[b], out_sh.at[dstv.at[b]], ssem.at[b],
                             add=True)

        def mwait_scat(b):
            pltpu.make_async_copy(mb.at[b], out_sh.at[pl.ds(0, CHS)],
                                  ssem.at[b]).wait()

        mload(0, 0)
        mload(1, 1)

        @pl.loop(0, NCHS, step=2)
        def _(k):
            for b in (0, 1):
                c = k + b
                mwait_load(b)
                mscat(b)

                @pl.when(c + 2 < NCHS)
                def _():
                    mwait_scat(b)
                    mload(c + 2, b)

        mwait_scat(0)
        mwait_scat(1)

        def eload(c, b):
            e0 = pl.multiple_of((cid * NS * NCHE + sid * NCHE + c) * CHS, CHS)
            pltpu.sync_copy(dst_hbm.at[pl.ds(e0, CHS)], dstv.at[b])
            for j in range(CHS // 16):
                dstv8[b, pl.ds(j * 16, 16)] = lax.shift_right_logical(
                    dstv[b, pl.ds(j * 16, 16)], 3)
            pltpu.async_copy(eb_hbm.at[pl.ds(e0, CHS)], ev.at[b], lsem.at[b])

        def ewait_load(b):
            pltpu.make_async_copy(eb_hbm.at[pl.ds(0, CHS)], ev.at[b],
                                  lsem.at[b]).wait()

        def escat(b):
            pltpu.async_copy(ev.at[b], s_sh.at[dstv8.at[b]], ssem.at[b],
                             add=True)

        def ewait_scat(b):
            pltpu.make_async_copy(ev.at[b], s_sh.at[pl.ds(0, CHS)],
                                  ssem.at[b]).wait()

        eload(0, 0)
        eload(1, 1)

        @pl.loop(0, NCHE, step=2)
        def _(k):
            for b in (0, 1):
                c = k + b
                ewait_load(b)
                escat(b)

                @pl.when(c + 2 < NCHE)
                def _():
                    ewait_scat(b)
                    eload(c + 2, b)

        ewait_scat(0)
        ewait_scat(1)

    @pl.when(cid == 0)
    def _():
        body(mlo_hbm)

    @pl.when(cid == 1)
    def _():
        body(mhi_hbm)

    plsc.subcore_barrier()

    def drain(dst_hbm_arr):
        @pl.when(sid < NS - 1)
        def _():
            pltpu.sync_copy(out_sh.at[pl.ds(rows0, NPS_A)],
                            dst_hbm_arr.at[pl.ds(rows0, NPS_A)])

        @pl.when(sid == NS - 1)
        def _():
            pltpu.sync_copy(out_sh.at[pl.ds((NS - 1) * NPS_A, NPS_L)],
                            dst_hbm_arr.at[pl.ds((NS - 1) * NPS_A, NPS_L)])

    @pl.when(cid == 0)
    def _():
        drain(outlo_hbm)

    @pl.when((cid == 0) & (sid == 0))
    def _():
        pltpu.sync_copy(s_sh, s0_hbm)

    @pl.when(cid == 1)
    def _():
        drain(outhi_hbm)

    @pl.when((cid == 1) & (sid == 0))
    def _():
        pltpu.sync_copy(s_sh, s1_hbm)


def _sc_scatter(mlo, mhi, eb, dst, z_row, z_s):
    mesh = plsc.VectorSubcoreMesh(core_axis_name="c", subcore_axis_name="s",
                                  num_cores=NC, num_subcores=NS)
    f = pl.kernel(
        _scatter_body,
        out_type=[jax.ShapeDtypeStruct((N, DH), _f32),
                  jax.ShapeDtypeStruct((N, DH), _f32),
                  jax.ShapeDtypeStruct((NS8, DKQ2), _f32),
                  jax.ShapeDtypeStruct((NS8, DKQ2), _f32)],
        mesh=mesh,
        scratch_types=[pltpu.VMEM_SHARED((N, DH), _f32),
                       pltpu.VMEM_SHARED((NS8, DKQ2), _f32),
                       pltpu.VMEM((2, CHS), jnp.int32),
                       pltpu.VMEM((2, CHS), jnp.int32),
                       pltpu.VMEM((2, CHS, DH), _f32),
                       pltpu.VMEM((2, CHS, DKQ2), _f32),
                       pltpu.SemaphoreType.DMA((2,)),
                       pltpu.SemaphoreType.DMA((2,))],
    )
    return f(mlo, mhi, eb, dst, z_row, z_s)


def _norm_body(lo, hi, s0, s1, x):
    inv = 1.0 / (s0[...][:, :1] + s1[...][:, :1] + 1e-9)
    x[:, :DH] = lo[...] * inv
    x[:, DH:] = hi[...] * inv


def _normalize(lo, hi, s0, s1, interpret=False):
    return pl.pallas_call(
        _norm_body,
        grid=(NBLK_N,),
        in_specs=[pl.BlockSpec((RB, DH), lambda i: (i, 0)),
                  pl.BlockSpec((RB, DH), lambda i: (i, 0)),
                  pl.BlockSpec((RB, DE), lambda i: (i, 0)),
                  pl.BlockSpec((RB, DE), lambda i: (i, 0))],
        out_specs=pl.BlockSpec((RB, DF), lambda i: (i, 0)),
        out_shape=jax.ShapeDtypeStruct((N, DF), _f32),
        interpret=interpret,
    )(lo, hi, s0, s1)


def _layer(x, ef_pad, src, dst, dst3, z_row, z_s, p):
    (Wk, Wv, Wrk1, brk1, Wrk2, brk2, Wrv1, brv1, Wrv2, brv2,
     Wq1, bq1, Wq2, bq2) = p
    KQ, V = _node_proj(x, Wk, Wv, Wq1, bq1.reshape(1, -1), Wq2,
                       bq2.reshape(1, -1))
    KQS, KQD, VS = _sc_gather(KQ, V, src, dst)
    Mlo, Mhi, eb = _edge_msg(ef_pad, KQS, KQD, VS, dst3,
                             Wrk1, brk1.reshape(1, -1), Wrk2, brk2.reshape(1, -1),
                             Wrv1, brv1.reshape(1, -1), Wrv2, brv2.reshape(1, -1))
    OL, OH, S0, S1 = _sc_scatter(Mlo, Mhi, eb, dst, z_row, z_s)
    return _normalize(OL, OH, S0.reshape(N, DE), S1.reshape(N, DE))


def kernel(node_features, edge_index, edge_features,
           Wk_0, Wv_0, Wrk1_0, brk1_0, Wrk2_0, brk2_0, Wrv1_0, brv1_0,
           Wrv2_0, brv2_0, Wq1_0, bq1_0, Wq2_0, bq2_0,
           Wk_1, Wv_1, Wrk1_1, brk1_1, Wrk2_1, brk2_1, Wrv1_1, brv1_1,
           Wrv2_1, brv2_1, Wq1_1, bq1_1, Wq2_1, bq2_1):
    ei_pad = jnp.pad(edge_index, ((0, 0), (0, EPAD - E)))
    src, dst = ei_pad[0], ei_pad[1]
    dst3 = dst.reshape(NEB, 1, EBLK)
    ef_pad = jnp.pad(edge_features, ((0, EPAD - E), (0, 0)))
    z_row = jnp.zeros((NPS_L, DH), _f32)
    z_s = jnp.zeros((NS8, DKQ2), _f32)
    layer_params = [
        [Wk_0, Wv_0, Wrk1_0, brk1_0, Wrk2_0, brk2_0, Wrv1_0, brv1_0,
         Wrv2_0, brv2_0, Wq1_0, bq1_0, Wq2_0, bq2_0],
        [Wk_1, Wv_1, Wrk1_1, brk1_1, Wrk2_1, brk2_1, Wrv1_1, brv1_1,
         Wrv2_1, brv2_1, Wq1_1, bq1_1, Wq2_1, bq2_1],
    ]
    x = node_features
    for p in layer_params:
        x = _layer(x, ef_pad, src, dst, dst3, z_row, z_s, p)
    return x
